# baseline (speedup 1.0000x reference)
import functools

import jax
import jax.numpy as jnp
from jax import lax
from jax.experimental import pallas as pl
from jax.experimental.pallas import tpu as pltpu
from jax.experimental.pallas import tpu_sc as plsc


@functools.lru_cache(maxsize=None)
def _build_gather(vocab, dim, n_rows):
    info = plsc.get_sparse_core_info()
    nc, ns = info.num_cores, info.num_subcores
    nw = nc * ns
    rows_per_w = n_rows // nw
    inflight = 16

    mesh = plsc.VectorSubcoreMesh(core_axis_name="c", subcore_axis_name="s")

    @functools.partial(
        pl.kernel,
        mesh=mesh,
        out_type=jax.ShapeDtypeStruct((n_rows, dim), jnp.float32),
        scratch_types=[
            pltpu.VMEM_SHARED((ns, rows_per_w), jnp.int32),
            pltpu.SMEM((rows_per_w,), jnp.int32),
            pltpu.SemaphoreType.DMA,
        ],
    )
    def gather_kernel(idx_hbm, table_hbm, out_hbm, idx_sp, ids_smem, sem):
        wid = lax.axis_index("s") * nc + lax.axis_index("c")
        sid = lax.axis_index("s")
        base = wid * rows_per_w
        pltpu.sync_copy(idx_hbm.at[wid], idx_sp.at[sid])
        pltpu.sync_copy(idx_sp.at[sid], ids_smem)

        handles = [None] * rows_per_w
        for r in range(rows_per_w):
            idx = ids_smem[r]
            handles[r] = pltpu.async_copy(
                table_hbm.at[pl.ds(idx, 1)],
                out_hbm.at[pl.ds(base + r, 1)],
                sem)
            if r >= inflight:
                handles[r - inflight].wait()
        for r in range(rows_per_w - inflight, rows_per_w):
            handles[r].wait()

    return gather_kernel, nw, rows_per_w


def kernel(input_ids, token_embedding):
    b, s = input_ids.shape
    vocab, dim = token_embedding.shape
    n_rows = b * s
    fn, nw, rows_per_w = _build_gather(vocab, dim, n_rows)
    idx = input_ids.reshape(nw, rows_per_w)
    out = fn(idx, token_embedding)
    return out.reshape(b, s, dim)


# speedup vs baseline: 30.6859x; 30.6859x over previous
import functools

import jax
import jax.numpy as jnp
from jax import lax
from jax.experimental import pallas as pl
from jax.experimental.pallas import tpu as pltpu
from jax.experimental.pallas import tpu_sc as plsc


@functools.lru_cache(maxsize=None)
def _build_gather(vocab, dim, batch, seq):
    info = plsc.get_sparse_core_info()
    nc, ns = info.num_cores, info.num_subcores
    nw = nc * ns
    n_rows = batch * seq
    rows_per_w = n_rows // nw
    chunk = 16
    n_chunks = rows_per_w // chunk
    nb = 3

    mesh = plsc.VectorSubcoreMesh(core_axis_name="c", subcore_axis_name="s")

    @functools.partial(
        pl.kernel,
        mesh=mesh,
        out_type=jax.ShapeDtypeStruct((n_rows, dim), jnp.float32),
        scratch_types=[
            pltpu.VMEM((n_chunks, chunk), jnp.int32),
        ]
        + [pltpu.VMEM((chunk, dim), jnp.float32) for _ in range(nb)]
        + [pltpu.SemaphoreType.DMA for _ in range(2 * nb + 1)],
    )
    def gather_kernel(idx_hbm, table_hbm, out_hbm, idx_v, *rest):
        bufs = rest[:nb]
        gsems = rest[nb:2 * nb]
        isem = rest[2 * nb]
        ssems = rest[2 * nb + 1:]
        wid = lax.axis_index("s") * nc + lax.axis_index("c")
        base = wid * rows_per_w
        w_per_b = seq // rows_per_w
        row = wid // w_per_b
        col = (wid % w_per_b) * rows_per_w
        i_handles = [
            pltpu.async_copy(
                idx_hbm.at[row, pl.ds(col + i * chunk, chunk)],
                idx_v.at[i], isem)
            for i in range(n_chunks)
        ]
        for h in i_handles:
            h.wait()

        g_handles = [None] * n_chunks
        s_handles = [None] * n_chunks
        for j in range(min(nb, n_chunks)):
            g_handles[j] = pltpu.async_copy(
                table_hbm.at[idx_v.at[j]], bufs[j], gsems[j])
        for i in range(n_chunks):
            if i >= 1 and i + nb - 1 < n_chunks:
                s_handles[i - 1].wait()
                j = i + nb - 1
                g_handles[j] = pltpu.async_copy(
                    table_hbm.at[idx_v.at[j]], bufs[j % nb], gsems[j % nb])
            g_handles[i].wait()
            s_handles[i] = pltpu.async_copy(
                bufs[i % nb], out_hbm.at[pl.ds(base + i * chunk, chunk)],
                ssems[i % nb])
        for i in range(max(0, n_chunks - nb), n_chunks):
            s_handles[i].wait()

    return gather_kernel


def kernel(input_ids, token_embedding):
    b, s = input_ids.shape
    vocab, dim = token_embedding.shape
    fn = _build_gather(vocab, dim, b, s)
    out = fn(input_ids, token_embedding)
    return out.reshape(b, s, dim)
